# baseline (speedup 1.0000x reference)
import functools

import jax
import jax.numpy as jnp
from jax import lax
from jax.experimental import pallas as pl
from jax.experimental.pallas import tpu as pltpu
from jax.experimental.pallas import tpu_sc as plsc

_NC = 2
_NS = 16
_NW = _NC * _NS
_BW = 128


def _emb_body(idxT_hbm, twide_hbm, posP_hbm, out_hbm,
              idx_v, pbuf_a, pbuf_b, obuf_a, obuf_b, pos_v,
              gsem_a, gsem_b, osem_a, osem_b, *, L, D):
    w = lax.axis_index("s") * _NC + lax.axis_index("c")
    b0 = w * _BW
    pltpu.sync_copy(idxT_hbm.at[:, pl.ds(b0, _BW)], idx_v)
    pltpu.sync_copy(posP_hbm, pos_v)

    iota16 = lax.iota(jnp.int32, 16)

    def gather(l, pbuf, sem):
        pltpu.async_copy(twide_hbm.at[idx_v.at[l]], pbuf, sem)

    def gwait(pbuf, sem):
        pltpu.make_async_copy(twide_hbm.at[idx_v.at[0]], pbuf, sem).wait()

    def flush(l, obuf, sem):
        pltpu.async_copy(obuf, out_hbm.at[l, :, pl.ds(b0, _BW)], sem)

    def owait(obuf, sem):
        pltpu.make_async_copy(obuf, out_hbm.at[0, :, pl.ds(b0, _BW)], sem).wait()

    def process(l, pbuf, obuf):
        pvs = []
        for c in range(D // 16):
            fi = (iota16 + (c * 16)) * L + l
            pvs.append(plsc.load_gather(
                pos_v, [lax.shift_right_logical(fi, 7), fi & 127]))

        def gbody(g, carry):
            rows = iota16 + g * 16
            for c in range(D // 16):
                for k in range(16):
                    rot = (iota16 + k) & 15
                    cols = rot + (c * 16)
                    vals = plsc.load_gather(pbuf, [rows, cols])
                    pos_scr = pvs[c].at[rot].get(mode="promise_in_bounds")
                    plsc.store_scatter(obuf, [cols, rows], vals + pos_scr)
            return carry

        lax.fori_loop(0, _BW // 16, gbody, 0)

    gather(0, pbuf_a, gsem_a)

    def step(ll, c):
        l0 = 2 * ll
        gwait(pbuf_a, gsem_a)
        gather(l0 + 1, pbuf_b, gsem_b)

        @pl.when(ll > 0)
        def _():
            owait(obuf_a, osem_a)

        process(l0, pbuf_a, obuf_a)
        flush(l0, obuf_a, osem_a)
        gwait(pbuf_b, gsem_b)

        @pl.when(ll < L // 2 - 1)
        def _():
            gather(l0 + 2, pbuf_a, gsem_a)

        @pl.when(ll > 0)
        def _():
            owait(obuf_b, osem_b)

        process(l0 + 1, pbuf_b, obuf_b)
        flush(l0 + 1, obuf_b, osem_b)
        return c

    lax.fori_loop(0, L // 2, step, 0)
    owait(obuf_a, osem_a)
    owait(obuf_b, osem_b)


def kernel(inputs, token_table, pos_table):
    B, L = inputs.shape
    V, D = token_table.shape
    assert pos_table.shape == (L, D)
    assert B == _NW * _BW and L % 2 == 0
    assert (L * D) % 128 == 0 and D % 16 == 0 and D <= 128 and 128 % D == 0

    idxT = jnp.transpose(inputs)
    if idxT.dtype != jnp.int32:
        idxT = idxT.astype(jnp.int32)
    twide = jnp.concatenate([token_table] * (128 // D), axis=1)
    posP = jnp.reshape(jnp.transpose(pos_table), (L * D // 128, 128))

    mesh = plsc.VectorSubcoreMesh(core_axis_name="c", subcore_axis_name="s")
    run = pl.kernel(
        functools.partial(_emb_body, L=L, D=D),
        mesh=mesh,
        compiler_params=pltpu.CompilerParams(needs_layout_passes=False),
        out_type=jax.ShapeDtypeStruct((L, D, B), jnp.float32),
        scratch_types=[
            pltpu.VMEM((L, _BW), jnp.int32),
            pltpu.VMEM((_BW, 128), jnp.float32),
            pltpu.VMEM((_BW, 128), jnp.float32),
            pltpu.VMEM((D, _BW), jnp.float32),
            pltpu.VMEM((D, _BW), jnp.float32),
            pltpu.VMEM((L * D // 128, 128), jnp.float32),
            pltpu.SemaphoreType.DMA,
            pltpu.SemaphoreType.DMA,
            pltpu.SemaphoreType.DMA,
            pltpu.SemaphoreType.DMA,
        ],
    )
    out3d = run(idxT, twide, posP)
    return jnp.transpose(out3d, (2, 0, 1))


# speedup vs baseline: 1.0039x; 1.0039x over previous
import functools

import jax
import jax.numpy as jnp
from jax import lax
from jax.experimental import pallas as pl
from jax.experimental.pallas import tpu as pltpu
from jax.experimental.pallas import tpu_sc as plsc

_NC = 2
_NS = 16
_NW = _NC * _NS
_BW = 128


def _emb_body(idxT_hbm, twide_hbm, posP_hbm, out_hbm,
              idx_v, pbuf_a, pbuf_b, obuf_a, obuf_b, pos_v,
              gsem_a, gsem_b, osem_a, osem_b, *, L, D):
    w = lax.axis_index("s") * _NC + lax.axis_index("c")
    b0 = w * _BW
    pltpu.sync_copy(idxT_hbm.at[:, pl.ds(b0, _BW)], idx_v)
    pltpu.sync_copy(posP_hbm, pos_v)

    iota16 = lax.iota(jnp.int32, 16)

    def gather(l, pbuf, sem):
        pltpu.async_copy(twide_hbm.at[idx_v.at[l]], pbuf, sem)

    def gwait(pbuf, sem):
        pltpu.make_async_copy(twide_hbm.at[idx_v.at[0]], pbuf, sem).wait()

    def flush(l, obuf, sem):
        pltpu.async_copy(obuf, out_hbm.at[l, :, pl.ds(b0, _BW)], sem)

    def owait(obuf, sem):
        pltpu.make_async_copy(obuf, out_hbm.at[0, :, pl.ds(b0, _BW)], sem).wait()

    def process(l, pbuf, obuf):
        pvs = []
        for c in range(D // 16):
            fi = (iota16 + (c * 16)) * L + l
            pvs.append(plsc.load_gather(
                pos_v, [lax.shift_right_logical(fi, 7), fi & 127]))

        def gbody(g, carry):
            rows = iota16 + g * 16
            for c in range(D // 16):
                for k in range(16):
                    rot = (iota16 + k) & 15
                    cols = rot + (c * 16)
                    vals = plsc.load_gather(pbuf, [rows, cols])
                    pos_scr = pvs[c].at[rot].get(mode="promise_in_bounds")
                    plsc.store_scatter(obuf, [cols, rows], vals + pos_scr)
            return carry

        lax.fori_loop(0, _BW // 16, gbody, 0)

    gather(0, pbuf_a, gsem_a)

    def step(ll, c):
        l0 = 2 * ll
        gwait(pbuf_a, gsem_a)
        gather(l0 + 1, pbuf_b, gsem_b)

        @pl.when(ll > 0)
        def _():
            owait(obuf_a, osem_a)

        process(l0, pbuf_a, obuf_a)
        flush(l0, obuf_a, osem_a)
        gwait(pbuf_b, gsem_b)

        @pl.when(ll < L // 2 - 1)
        def _():
            gather(l0 + 2, pbuf_a, gsem_a)

        @pl.when(ll > 0)
        def _():
            owait(obuf_b, osem_b)

        process(l0 + 1, pbuf_b, obuf_b)
        flush(l0 + 1, obuf_b, osem_b)
        return c

    lax.fori_loop(0, L // 2, step, 0)
    owait(obuf_a, osem_a)
    owait(obuf_b, osem_b)


def kernel(inputs, token_table, pos_table):
    B, L = inputs.shape
    V, D = token_table.shape
    assert pos_table.shape == (L, D)
    assert B == _NW * _BW and L % 2 == 0
    assert (L * D) % 128 == 0 and D % 16 == 0 and D <= 128 and 128 % D == 0

    idxT = jnp.transpose(inputs)
    if idxT.dtype != jnp.int32:
        idxT = idxT.astype(jnp.int32)
    twide = jnp.reshape(
        jnp.broadcast_to(token_table[:, None, :], (V, 128 // D, D)), (V, 128))
    posP = jnp.reshape(jnp.transpose(pos_table), (L * D // 128, 128))

    mesh = plsc.VectorSubcoreMesh(core_axis_name="c", subcore_axis_name="s")
    run = pl.kernel(
        functools.partial(_emb_body, L=L, D=D),
        mesh=mesh,
        compiler_params=pltpu.CompilerParams(needs_layout_passes=False),
        out_type=jax.ShapeDtypeStruct((L, D, B), jnp.float32),
        scratch_types=[
            pltpu.VMEM((L, _BW), jnp.int32),
            pltpu.VMEM((_BW, 128), jnp.float32),
            pltpu.VMEM((_BW, 128), jnp.float32),
            pltpu.VMEM((D, _BW), jnp.float32),
            pltpu.VMEM((D, _BW), jnp.float32),
            pltpu.VMEM((L * D // 128, 128), jnp.float32),
            pltpu.SemaphoreType.DMA,
            pltpu.SemaphoreType.DMA,
            pltpu.SemaphoreType.DMA,
            pltpu.SemaphoreType.DMA,
        ],
    )
    out3d = run(idxT, twide, posP)
    return jnp.transpose(out3d, (2, 0, 1))


# speedup vs baseline: 1.0507x; 1.0466x over previous
import functools

import jax
import jax.numpy as jnp
from jax import lax
from jax.experimental import pallas as pl
from jax.experimental.pallas import tpu as pltpu
from jax.experimental.pallas import tpu_sc as plsc

_NC = 2
_NS = 16
_NW = _NC * _NS
_BW = 128


def _emb_body(idxT_hbm, twide_hbm, posP_hbm, out_hbm,
              idx_v, pbuf_a, pbuf_b, obuf_a, obuf_b, pos_v,
              gsem_a, gsem_b, osem_a, osem_b, *, L, D):
    w = lax.axis_index("s") * _NC + lax.axis_index("c")
    b0 = w * _BW
    pltpu.sync_copy(idxT_hbm.at[:, pl.ds(b0, _BW)], idx_v)
    pltpu.sync_copy(posP_hbm, pos_v)

    iota16 = lax.iota(jnp.int32, 16)

    def gather(l, pbuf, sem):
        pltpu.async_copy(twide_hbm.at[idx_v.at[l]], pbuf, sem)

    def gwait(pbuf, sem):
        pltpu.make_async_copy(twide_hbm.at[idx_v.at[0]], pbuf, sem).wait()

    def flush(l, obuf, sem):
        pltpu.async_copy(obuf, out_hbm.at[l, :, w, :, :], sem)

    def owait(obuf, sem):
        pltpu.make_async_copy(obuf, out_hbm.at[0, :, w, :, :], sem).wait()

    def process(l, pbuf, obuf):
        pvs = []
        for c in range(D // 16):
            fi = (iota16 + (c * 16)) * L + l
            pvs.append(plsc.load_gather(
                pos_v, [lax.shift_right_logical(fi, 7), fi & 127]))

        def gbody(g, carry):
            rows = iota16 + g * 16
            for c in range(D // 16):
                for k in range(16):
                    rot = (iota16 + k) & 15
                    cols = rot + (c * 16)
                    vals = plsc.load_gather(pbuf, [rows, cols])
                    pos_scr = pvs[c].at[rot].get(mode="promise_in_bounds")
                    plsc.store_scatter(
                        obuf, [lax.shift_right_logical(cols, 3), cols & 7, rows],
                        vals + pos_scr)
            return carry

        lax.fori_loop(0, _BW // 16, gbody, 0)

    gather(0, pbuf_a, gsem_a)

    def step(ll, c):
        l0 = 2 * ll
        gwait(pbuf_a, gsem_a)
        gather(l0 + 1, pbuf_b, gsem_b)

        @pl.when(ll > 0)
        def _():
            owait(obuf_a, osem_a)

        process(l0, pbuf_a, obuf_a)
        flush(l0, obuf_a, osem_a)
        gwait(pbuf_b, gsem_b)

        @pl.when(ll < L // 2 - 1)
        def _():
            gather(l0 + 2, pbuf_a, gsem_a)

        @pl.when(ll > 0)
        def _():
            owait(obuf_b, osem_b)

        process(l0 + 1, pbuf_b, obuf_b)
        flush(l0 + 1, obuf_b, osem_b)
        return c

    lax.fori_loop(0, L // 2, step, 0)
    owait(obuf_a, osem_a)
    owait(obuf_b, osem_b)


def kernel(inputs, token_table, pos_table):
    B, L = inputs.shape
    V, D = token_table.shape
    assert pos_table.shape == (L, D)
    assert B == _NW * _BW and L % 2 == 0
    assert (L * D) % 128 == 0 and D % 16 == 0 and D <= 128 and 128 % D == 0

    idxT = jnp.transpose(inputs)
    if idxT.dtype != jnp.int32:
        idxT = idxT.astype(jnp.int32)
    posP = jnp.reshape(jnp.transpose(pos_table), (L * D // 128, 128))

    mesh = plsc.VectorSubcoreMesh(core_axis_name="c", subcore_axis_name="s")
    run = pl.kernel(
        functools.partial(_emb_body, L=L, D=D),
        mesh=mesh,
        compiler_params=pltpu.CompilerParams(
            needs_layout_passes=False, use_tc_tiling_on_sc=False),
        out_type=jax.ShapeDtypeStruct((L, D // 8, B // 128, 8, 128), jnp.float32),
        scratch_types=[
            pltpu.VMEM((L, _BW), jnp.int32),
            pltpu.VMEM((_BW, 64), jnp.float32),
            pltpu.VMEM((_BW, 64), jnp.float32),
            pltpu.VMEM((D // 8, 8, _BW), jnp.float32),
            pltpu.VMEM((D // 8, 8, _BW), jnp.float32),
            pltpu.VMEM((L * D // 128, 128), jnp.float32),
            pltpu.SemaphoreType.DMA,
            pltpu.SemaphoreType.DMA,
            pltpu.SemaphoreType.DMA,
            pltpu.SemaphoreType.DMA,
        ],
    )
    out5 = run(idxT, token_table, posP)
    return jnp.reshape(jnp.transpose(out5, (2, 4, 0, 1, 3)), (B, L, D))


# speedup vs baseline: 1.0553x; 1.0044x over previous
import functools

import jax
import jax.numpy as jnp
from jax import lax
from jax.experimental import pallas as pl
from jax.experimental.pallas import tpu as pltpu
from jax.experimental.pallas import tpu_sc as plsc

_NC = 2
_NS = 16
_NW = _NC * _NS
_BW = 128


def _emb_body(idxT_hbm, twide_hbm, posP_hbm, out_hbm,
              idx_v, pbuf_a, pbuf_b, pbuf_c, pbuf_d, obuf_a, obuf_b, pos_v,
              gsem_a, gsem_b, gsem_c, gsem_d, osem_a, osem_b, *, L, D):
    w = lax.axis_index("s") * _NC + lax.axis_index("c")
    b0 = w * _BW
    pltpu.sync_copy(idxT_hbm.at[:, pl.ds(b0, _BW)], idx_v)
    pltpu.sync_copy(posP_hbm, pos_v)

    iota16 = lax.iota(jnp.int32, 16)

    def gather(l, pbuf, sem):
        pltpu.async_copy(twide_hbm.at[idx_v.at[l]], pbuf, sem)

    def gwait(pbuf, sem):
        pltpu.make_async_copy(twide_hbm.at[idx_v.at[0]], pbuf, sem).wait()

    def flush(l, obuf, sem):
        pltpu.async_copy(obuf, out_hbm.at[l, :, w, :, :], sem)

    def owait(obuf, sem):
        pltpu.make_async_copy(obuf, out_hbm.at[0, :, w, :, :], sem).wait()

    def process(l, pbuf, obuf):
        pvs = []
        for c in range(D // 16):
            fi = (iota16 + (c * 16)) * L + l
            pvs.append(plsc.load_gather(
                pos_v, [lax.shift_right_logical(fi, 7), fi & 127]))

        def gbody(g, carry):
            rows = iota16 + g * 16
            for c in range(D // 16):
                for k in range(16):
                    rot = (iota16 + k) & 15
                    cols = rot + (c * 16)
                    vals = plsc.load_gather(pbuf, [rows, cols])
                    pos_scr = pvs[c].at[rot].get(mode="promise_in_bounds")
                    plsc.store_scatter(
                        obuf, [lax.shift_right_logical(cols, 3), cols & 7, rows],
                        vals + pos_scr)
            return carry

        lax.fori_loop(0, _BW // 16, gbody, 0)

    gather(0, pbuf_a, gsem_a)
    gather(1, pbuf_b, gsem_b)

    def quad(q, c):
        l0 = 4 * q
        for (j, pbuf, gsem, nbuf, nsem, obuf, osem) in (
            (0, pbuf_a, gsem_a, pbuf_c, gsem_c, obuf_a, osem_a),
            (1, pbuf_b, gsem_b, pbuf_d, gsem_d, obuf_b, osem_b),
            (2, pbuf_c, gsem_c, pbuf_a, gsem_a, obuf_a, osem_a),
            (3, pbuf_d, gsem_d, pbuf_b, gsem_b, obuf_b, osem_b),
        ):
            gwait(pbuf, gsem)

            @pl.when(l0 + j + 2 < L)
            def _():
                gather(l0 + j + 2, nbuf, nsem)

            @pl.when(l0 + j > 1)
            def _():
                owait(obuf, osem)

            process(l0 + j, pbuf, obuf)
            flush(l0 + j, obuf, osem)
        return c

    lax.fori_loop(0, L // 4, quad, 0)
    owait(obuf_a, osem_a)
    owait(obuf_b, osem_b)


def kernel(inputs, token_table, pos_table):
    B, L = inputs.shape
    V, D = token_table.shape
    assert pos_table.shape == (L, D)
    assert B == _NW * _BW and L % 4 == 0
    assert (L * D) % 128 == 0 and D % 16 == 0 and D <= 128 and 128 % D == 0

    idxT = jnp.transpose(inputs)
    if idxT.dtype != jnp.int32:
        idxT = idxT.astype(jnp.int32)
    posP = jnp.reshape(jnp.transpose(pos_table), (L * D // 128, 128))

    mesh = plsc.VectorSubcoreMesh(core_axis_name="c", subcore_axis_name="s")
    run = pl.kernel(
        functools.partial(_emb_body, L=L, D=D),
        mesh=mesh,
        compiler_params=pltpu.CompilerParams(
            needs_layout_passes=False, use_tc_tiling_on_sc=False),
        out_type=jax.ShapeDtypeStruct((L, D // 8, B // 128, 8, 128), jnp.float32),
        scratch_types=[
            pltpu.VMEM((L, _BW), jnp.int32),
            pltpu.VMEM((_BW, 64), jnp.float32),
            pltpu.VMEM((_BW, 64), jnp.float32),
            pltpu.VMEM((_BW, 64), jnp.float32),
            pltpu.VMEM((_BW, 64), jnp.float32),
            pltpu.VMEM((D // 8, 8, _BW), jnp.float32),
            pltpu.VMEM((D // 8, 8, _BW), jnp.float32),
            pltpu.VMEM((L * D // 128, 128), jnp.float32),
            pltpu.SemaphoreType.DMA,
            pltpu.SemaphoreType.DMA,
            pltpu.SemaphoreType.DMA,
            pltpu.SemaphoreType.DMA,
            pltpu.SemaphoreType.DMA,
            pltpu.SemaphoreType.DMA,
        ],
    )
    out5 = run(idxT, token_table, posP)
    return jnp.reshape(jnp.transpose(out5, (2, 4, 0, 1, 3)), (B, L, D))
